# baseline (speedup 1.0000x reference)
import functools

import jax
import jax.numpy as jnp
from jax import lax
from jax.experimental import pallas as pl
from jax.experimental.pallas import tpu as pltpu
from jax.experimental.pallas import tpu_sc as plsc

B = 16384
N_NUM = 13
N_CAT = 26
H = 128
N_FIELDS = N_NUM + N_CAT

NC = 2
NS = 16
NW = NC * NS

ROWS_PER_W = B // NW
CHUNK = 128
NCH = ROWS_PER_W // CHUNK
NBUF = 4


@functools.partial(
    pl.kernel,
    mesh=plsc.VectorSubcoreMesh(core_axis_name="c", subcore_axis_name="s"),
    out_type=jax.ShapeDtypeStruct((B * N_FIELDS, H), jnp.float32),
    scratch_types=[
        pltpu.VMEM((N_CAT * ROWS_PER_W,), jnp.int32),
        pltpu.VMEM((CHUNK, H), jnp.float32),
        pltpu.VMEM((CHUNK, H), jnp.float32),
        pltpu.VMEM((CHUNK, H), jnp.float32),
        pltpu.VMEM((CHUNK, H), jnp.float32),
        pltpu.SemaphoreType.DMA,
        pltpu.SemaphoreType.DMA,
        pltpu.SemaphoreType.DMA,
        pltpu.SemaphoreType.DMA,
        pltpu.SemaphoreType.DMA,
        pltpu.SemaphoreType.DMA,
        pltpu.SemaphoreType.DMA,
        pltpu.SemaphoreType.DMA,
    ],
)
def _gather_sc_kernel(
    catidx_hbm, table_hbm, out_hbm,
    idx_v, buf0, buf1, buf2, buf3,
    g0, g1, g2, g3, s0, s1, s2, s3,
):
    bufs = (buf0, buf1, buf2, buf3)
    gsem = (g0, g1, g2, g3)
    ssem = (s0, s1, s2, s3)
    wid = lax.axis_index("s") * NC + lax.axis_index("c")
    wb = wid * ROWS_PER_W

    def idx_ref(j, b):
        return idx_v.at[pl.ds(j * ROWS_PER_W + b * CHUNK, CHUNK)]

    pltpu.async_copy(
        catidx_hbm.at[pl.ds(wb, ROWS_PER_W)], idx_v.at[pl.ds(0, ROWS_PER_W)], g0
    )
    for j in range(1, N_CAT):
        pltpu.async_copy(
            catidx_hbm.at[pl.ds(j * B + wb, ROWS_PER_W)],
            idx_v.at[pl.ds(j * ROWS_PER_W, ROWS_PER_W)],
            s0,
        )
    pltpu.make_async_copy(
        catidx_hbm.at[pl.ds(0, ROWS_PER_W)], idx_v.at[pl.ds(0, ROWS_PER_W)], g0
    ).wait()
    for b in range(NBUF):
        pltpu.async_copy(table_hbm.at[idx_ref(0, b)], bufs[b], gsem[b])
    for j in range(1, N_CAT):
        pltpu.make_async_copy(
            catidx_hbm.at[pl.ds(0, ROWS_PER_W)], idx_v.at[pl.ds(0, ROWS_PER_W)], s0
        ).wait()

    def main_body(j, carry):
        out_base = (N_NUM + j) * B + wb
        for b in range(NBUF):
            pltpu.make_async_copy(table_hbm.at[idx_ref(0, b)], bufs[b], gsem[b]).wait()
            pltpu.async_copy(
                bufs[b], out_hbm.at[pl.ds(out_base + b * CHUNK, CHUNK)], ssem[b]
            )
        for b in range(NBUF):
            pltpu.make_async_copy(bufs[b], out_hbm.at[pl.ds(0, CHUNK)], ssem[b]).wait()

            @pl.when(j < N_CAT - 1)
            def _():
                pltpu.async_copy(table_hbm.at[idx_ref(j + 1, b)], bufs[b], gsem[b])

        return carry

    lax.fori_loop(0, N_CAT, main_body, 0, unroll=False)


BLK = 2048
NBB = B // BLK


def _num_tc_body(xn_ref, emb_ref, alias_ref, out_ref):
    del alias_ref
    bb = pl.program_id(1)
    x = xn_ref[0, 0, pl.ds(bb * BLK, BLK)]
    out_ref[...] = x[:, None] * emb_ref[0, 0, :]


_num_tc_kernel = pl.pallas_call(
    _num_tc_body,
    grid=(N_NUM, NBB),
    in_specs=[
        pl.BlockSpec((1, 1, B), lambda k, b: (k, 0, 0)),
        pl.BlockSpec((1, 1, H), lambda k, b: (k, 0, 0)),
        pl.BlockSpec(memory_space=pl.ANY),
    ],
    out_specs=pl.BlockSpec((BLK, H), lambda k, b: (k * NBB + b, 0)),
    out_shape=jax.ShapeDtypeStruct((B * N_FIELDS, H), jnp.float32),
    input_output_aliases={2: 0},
)


def kernel(X_numerical, X_categorical, word_embeddings, num_embeddings):
    catidx = jnp.transpose(X_categorical.astype(jnp.int32)).reshape(-1)
    sc_out = _gather_sc_kernel(catidx, word_embeddings)
    xnum_t = jnp.transpose(X_numerical).reshape(N_NUM, 1, B)
    emb = num_embeddings.reshape(N_NUM, 1, H)
    out = _num_tc_kernel(xnum_t, emb, sc_out)
    out = out.reshape(N_FIELDS, B, H)
    return jnp.transpose(out, (1, 0, 2))


# speedup vs baseline: 1.0946x; 1.0946x over previous
import functools

import jax
import jax.numpy as jnp
from jax import lax
from jax.experimental import pallas as pl
from jax.experimental.pallas import tpu as pltpu
from jax.experimental.pallas import tpu_sc as plsc

B = 16384
N_NUM = 13
N_CAT = 26
H = 128
N_FIELDS = N_NUM + N_CAT

NC = 2
NS = 16
NW = NC * NS

ROWS_PER_W = B // NW
CHUNK = 128
NCH = ROWS_PER_W // CHUNK
NBUF = 4


@functools.partial(
    pl.kernel,
    mesh=plsc.VectorSubcoreMesh(core_axis_name="c", subcore_axis_name="s"),
    out_type=jax.ShapeDtypeStruct((B * N_FIELDS, H), jnp.float32),
    scratch_types=[
        pltpu.VMEM((N_CAT * ROWS_PER_W,), jnp.int32),
        pltpu.VMEM((N_NUM * ROWS_PER_W + 16,), jnp.float32),
        pltpu.VMEM((N_NUM * H,), jnp.float32),
        pltpu.VMEM((CHUNK, H), jnp.float32),
        pltpu.VMEM((CHUNK, H), jnp.float32),
        pltpu.VMEM((CHUNK, H), jnp.float32),
        pltpu.VMEM((CHUNK, H), jnp.float32),
        pltpu.VMEM((64, H), jnp.float32),
        pltpu.VMEM((64, H), jnp.float32),
        pltpu.VMEM((64, H), jnp.float32),
        pltpu.VMEM((64, H), jnp.float32),
        pltpu.SemaphoreType.DMA,
        pltpu.SemaphoreType.DMA,
        pltpu.SemaphoreType.DMA,
        pltpu.SemaphoreType.DMA,
        pltpu.SemaphoreType.DMA,
        pltpu.SemaphoreType.DMA,
        pltpu.SemaphoreType.DMA,
        pltpu.SemaphoreType.DMA,
        pltpu.SemaphoreType.DMA,
        pltpu.SemaphoreType.DMA,
        pltpu.SemaphoreType.DMA,
        pltpu.SemaphoreType.DMA,
    ],
)
def _embed_kernel(
    xnum_hbm, catidx_hbm, table_hbm, emb_hbm, out_hbm,
    idx_v, xnum_v, emb_v, buf0, buf1, buf2, buf3, nbuf0, nbuf1, nbuf2, nbuf3,
    g0, g1, g2, g3, s0, s1, s2, s3, n0, n1, n2, n3,
):
    bufs = (buf0, buf1, buf2, buf3)
    nbufs = (nbuf0, nbuf1, nbuf2, nbuf3)
    gsem = (g0, g1, g2, g3)
    ssem = (s0, s1, s2, s3)
    nsem = (n0, n1, n2, n3)
    wid = lax.axis_index("s") * NC + lax.axis_index("c")
    wb = wid * ROWS_PER_W

    def idx_ref(j, b):
        return idx_v.at[pl.ds(j * ROWS_PER_W + b * CHUNK, CHUNK)]

    pltpu.async_copy(
        catidx_hbm.at[pl.ds(wb, ROWS_PER_W)], idx_v.at[pl.ds(0, ROWS_PER_W)], g0
    )
    for j in range(1, N_CAT):
        pltpu.async_copy(
            catidx_hbm.at[pl.ds(j * B + wb, ROWS_PER_W)],
            idx_v.at[pl.ds(j * ROWS_PER_W, ROWS_PER_W)],
            s0,
        )
    for k in range(N_NUM):
        pltpu.async_copy(
            xnum_hbm.at[pl.ds(k * B + wb, ROWS_PER_W)],
            xnum_v.at[pl.ds(k * ROWS_PER_W, ROWS_PER_W)],
            s1,
        )
    pltpu.async_copy(emb_hbm, emb_v, s2)
    pltpu.make_async_copy(
        catidx_hbm.at[pl.ds(0, ROWS_PER_W)], idx_v.at[pl.ds(0, ROWS_PER_W)], g0
    ).wait()
    for b in range(NBUF):
        pltpu.async_copy(table_hbm.at[idx_ref(0, b)], bufs[b], gsem[b])
    for j in range(1, N_CAT):
        pltpu.make_async_copy(
            catidx_hbm.at[pl.ds(0, ROWS_PER_W)], idx_v.at[pl.ds(0, ROWS_PER_W)], s0
        ).wait()
    for k in range(N_NUM):
        pltpu.make_async_copy(
            xnum_hbm.at[pl.ds(0, ROWS_PER_W)], xnum_v.at[pl.ds(0, ROWS_PER_W)], s1
        ).wait()
    pltpu.make_async_copy(emb_hbm, emb_v, s2).wait()

    def num_chunk(j, par):
        c = j * 4 + par
        k = c // 8
        bo = c % 8

        @pl.when(j >= 1)
        def _():
            pltpu.make_async_copy(
                nbufs[par], out_hbm.at[pl.ds(0, 64)], nsem[par]
            ).wait()

        base = k * ROWS_PER_W + bo * 64
        evecs = [emb_v[pl.ds(k * H + h * 16, 16)] for h in range(H // 16)]
        for i in range(64 // 16):
            xvec = xnum_v[pl.ds(base + i * 16, 16)]
            for l in range(16):
                x = xvec[l]
                r = i * 16 + l
                for h in range(H // 16):
                    nbufs[par][r, pl.ds(h * 16, 16)] = x * evecs[h]
        pltpu.async_copy(
            nbufs[par], out_hbm.at[pl.ds(k * B + wb + bo * 64, 64)], nsem[par]
        )

    def drain_and_refill(j, b):
        pltpu.make_async_copy(bufs[b], out_hbm.at[pl.ds(0, CHUNK)], ssem[b]).wait()

        @pl.when(j < N_CAT - 1)
        def _():
            pltpu.async_copy(table_hbm.at[idx_ref(j + 1, b)], bufs[b], gsem[b])

    def main_body(j, carry):
        out_base = (N_NUM + j) * B + wb
        for b in range(NBUF):
            pltpu.make_async_copy(table_hbm.at[idx_ref(0, b)], bufs[b], gsem[b]).wait()
            pltpu.async_copy(
                bufs[b], out_hbm.at[pl.ds(out_base + b * CHUNK, CHUNK)], ssem[b]
            )
        for q in range(4):
            num_chunk(j, q)
            drain_and_refill(j, q)
        return carry

    lax.fori_loop(0, N_CAT, main_body, 0, unroll=False)
    for par in range(4):
        pltpu.make_async_copy(nbufs[par], out_hbm.at[pl.ds(0, 64)], nsem[par]).wait()


def kernel(X_numerical, X_categorical, word_embeddings, num_embeddings):
    xnum = jnp.transpose(X_numerical).reshape(-1)
    catidx = jnp.transpose(X_categorical.astype(jnp.int32)).reshape(-1)
    emb = num_embeddings.reshape(-1)
    out = _embed_kernel(xnum, catidx, word_embeddings, emb)
    out = out.reshape(N_FIELDS, B, H)
    return jnp.transpose(out, (1, 0, 2))
